# Initial kernel scaffold; baseline (speedup 1.0000x reference)
import jax
import jax.numpy as jnp
from jax.experimental import pallas as pl


def kernel(elem_weights, elem_fea, sym_fea, self_idx, nbr_idx, cry_elem_idx, aug_cry_idx, params):
    raise NotImplementedError("write your pallas kernel here")


# speedup vs baseline: 1.0003x; 1.0003x over previous
import jax
import jax.numpy as jnp
from jax.experimental import pallas as pl

N_CRY = 1250
N_AUG = 160


def _simple_net(p, x):
    for W, b in p["hidden"]:
        x = jax.nn.leaky_relu(x @ W + b, negative_slope=0.01)
    W, b = p["out"]
    return x @ W + b


def _attn_pool(p, x, index, weights, num_segments):
    gate = _simple_net(p["gate"], x)
    gmax = jax.ops.segment_max(gate, index, num_segments=num_segments)
    gate = gate - gmax[index]
    gate = (weights ** p["pow"]) * jnp.exp(gate)
    denom = jax.ops.segment_sum(gate, index, num_segments=num_segments)
    gate = gate / (denom[index] + 1e-10)
    msg = _simple_net(p["message"], x)
    return jax.ops.segment_sum(gate * msg, index, num_segments=num_segments)


def _message_layer(layer, elem_weights, elem_fea, self_idx, nbr_idx):
    n = elem_fea.shape[0]
    nbr_w = elem_weights[nbr_idx]
    fea = jnp.concatenate([elem_fea[self_idx], elem_fea[nbr_idx]], axis=1)
    heads = [_attn_pool(p, fea, self_idx, nbr_w, n) for p in layer["heads"]]
    return jnp.mean(jnp.stack(heads), axis=0) + elem_fea


def _div_kernel(s_ref, c_ref, o_ref):
    o_ref[...] = s_ref[...] / jnp.maximum(c_ref[...], 1.0)


def kernel(elem_weights, elem_fea, sym_fea, self_idx, nbr_idx, cry_elem_idx, aug_cry_idx, params):
    W, b = params["elem_embed"]
    elem = elem_fea @ W + b
    W, b = params["sym_embed"]
    sym = jnp.concatenate([sym_fea, elem_weights], axis=1) @ W + b
    fea = jnp.concatenate([elem, sym], axis=1)
    for layer in params["graphs"]:
        fea = _message_layer(layer, elem_weights, fea, self_idx, nbr_idx)
    heads = [_attn_pool(p, fea, cry_elem_idx, elem_weights, N_CRY) for p in params["cry_pool"]]
    cry = jnp.mean(jnp.stack(heads), axis=0)
    summed = jax.ops.segment_sum(cry, aug_cry_idx, num_segments=N_AUG)
    counts = jax.ops.segment_sum(jnp.ones((cry.shape[0],), cry.dtype), aug_cry_idx, num_segments=N_AUG)
    counts2 = jnp.broadcast_to(counts[:, None], summed.shape)
    return pl.pallas_call(
        _div_kernel,
        out_shape=jax.ShapeDtypeStruct(summed.shape, summed.dtype),
    )(summed, counts2)


# speedup vs baseline: 1.1457x; 1.1453x over previous
import functools
import jax
import jax.numpy as jnp
from jax import lax
from jax.experimental import pallas as pl
from jax.experimental.pallas import tpu as pltpu
from jax.experimental.pallas import tpu_sc as plsc

INTERPRET = False

N_NODES = 10000
N2 = 10240
NPT = 320
N_EDGES = 160000
CHUNK = 2000
NCH = N_EDGES // CHUNK
GCAP = 163840
N_CRY = 1250
NCRY2 = 1280
CPT = 40
N_AUG = 160
F = 64
H = 256
NT = 32
L = 16
NEG = -3.0e38


def _mesh():
    return plsc.VectorSubcoreMesh(core_axis_name="c", subcore_axis_name="s")


_SC_PARAMS = pltpu.CompilerParams(needs_layout_passes=False)


def _zero2d(ref, rows):
    def zrow(i, _):
        for k in range(H // L):
            ref[i, pl.ds(k * L, L)] = jnp.zeros((L,), jnp.float32)
        return 0
    lax.fori_loop(0, rows, zrow, 0)


def _fill1d(ref, n, val):
    def zb(i, _):
        ref[pl.ds(i * L, L)] = jnp.full((L,), val, jnp.float32)
        return 0
    lax.fori_loop(0, n // L, zb, 0)


def _edge_body(ag_h, bg_h, am_h, bm_h, wg2_h, s_h, n_h, wp_h,
               S_h, den_h, glog_h,
               big, s_in, n_in, wl_s, wl_n, gbuf, glogb, wg2b, gmaxb, denb,
               wpv, sem, sem2):
    cix = lax.axis_index("c")
    six = lax.axis_index("s")
    wid = cix * 16 + six
    n0 = wid * NPT
    iota = lax.iota(jnp.int32, L)

    pltpu.sync_copy(wg2_h, wg2b)
    pltpu.sync_copy(wp_h, wpv)
    pltpu.sync_copy(ag_h.at[pl.ds(pl.multiple_of(n0, 16), NPT)], big)
    _fill1d(gmaxb, NPT, NEG)
    _fill1d(denb, NPT, 0.0)

    def compact(c):
        pltpu.sync_copy(s_h.at[pl.ds(pl.multiple_of(c * CHUNK, 16), CHUNK)], s_in)
        pltpu.sync_copy(n_h.at[pl.ds(pl.multiple_of(c * CHUNK, 16), CHUNK)], n_in)

        def cbody(i, off):
            vs = s_in[pl.ds(i * L, L)]
            vn = n_in[pl.ds(i * L, L)]
            m = (vs >= n0) & (vs < n0 + NPT)
            plsc.store_compressed(wl_s.at[pl.ds(off, L)], vs, mask=m)
            plsc.store_compressed(wl_n.at[pl.ds(off, L)], vn, mask=m)
            cnt = jnp.sum(jnp.where(m, jnp.full((L,), 1.0, jnp.float32),
                                    jnp.full((L,), 0.0, jnp.float32)))
            return off + cnt.astype(jnp.int32)

        off = lax.fori_loop(0, CHUNK // L, cbody, 0)
        wl_s[pl.ds(off, L)] = jnp.full((L,), n0, jnp.int32)
        wl_n[pl.ds(off, L)] = jnp.zeros((L,), jnp.int32)
        return off

    def p1chunk(c, goff):
        off = compact(c)
        ngrp = (off + L - 1) // L

        def gbody(g, _):
            pltpu.async_copy(bg_h.at[wl_n.at[pl.ds(g * L, L)]], gbuf.at[0],
                             sem).wait()
            sv = wl_s[pl.ds(g * L, L)] - n0
            vglog = jnp.zeros((L,), jnp.float32)
            for j in range(L):
                sj = sv[j]
                dacc = jnp.zeros((L,), jnp.float32)
                for k in range(H // L):
                    va = big[sj, pl.ds(k * L, L)]
                    vb = gbuf[0, j, pl.ds(k * L, L)]
                    hh = va + vb
                    r = jnp.maximum(hh, 0.01 * hh)
                    dacc = dacc + r * wg2b[pl.ds(k * L, L)]
                gl = jnp.sum(dacc)

                @pl.when(g * L + j < off)
                def _():
                    spl = jnp.full((L,), sj, jnp.int32)
                    old = plsc.load_gather(gmaxb, [spl])
                    plsc.store_scatter(
                        gmaxb, [spl],
                        jnp.maximum(old, jnp.full((L,), gl, jnp.float32)))

                vglog = jnp.where(iota == j, gl, vglog)
            glogb[pl.ds(g * L, L)] = vglog
            return 0

        lax.fori_loop(0, ngrp, gbody, 0)
        pltpu.sync_copy(glogb, glog_h.at[pl.ds(pl.multiple_of(wid * GCAP + goff, 16), CHUNK + L)])
        return goff + ngrp * L

    lax.fori_loop(0, NCH, p1chunk, 0)

    _zero2d(big, NPT)

    def p2chunk(c, goff):
        off = compact(c)
        ngrp = (off + L - 1) // L
        pltpu.sync_copy(glog_h.at[pl.ds(pl.multiple_of(wid * GCAP + goff, 16), CHUNK + L)], glogb)

        def gbody(g, _):
            cpa = pltpu.async_copy(am_h.at[wl_s.at[pl.ds(g * L, L)]],
                                   gbuf.at[0], sem)
            cpb = pltpu.async_copy(bm_h.at[wl_n.at[pl.ds(g * L, L)]],
                                   gbuf.at[1], sem2)
            cpa.wait()
            cpb.wait()
            sv = wl_s[pl.ds(g * L, L)] - n0
            nv = wl_n[pl.ds(g * L, L)]
            gl16 = glogb[pl.ds(g * L, L)]
            gm16 = plsc.load_gather(gmaxb, [sv])
            wp16 = plsc.load_gather(wpv, [nv])
            valid = (g * L + iota) < off
            p16 = jnp.where(valid, wp16 * jnp.exp(gl16 - gm16),
                            jnp.zeros((L,), jnp.float32))
            plsc.addupdate_scatter(denb, [sv], p16)
            for j in range(L):
                sj = sv[j]
                pj = jnp.full((L,), p16[j], jnp.float32)
                for k in range(H // L):
                    am = gbuf[0, j, pl.ds(k * L, L)]
                    bm = gbuf[1, j, pl.ds(k * L, L)]
                    hh = am + bm
                    r = jnp.maximum(hh, 0.01 * hh)
                    plsc.addupdate(big.at[sj, pl.ds(k * L, L)], r * pj)
            return 0

        lax.fori_loop(0, ngrp, gbody, 0)
        return goff + ngrp * L

    lax.fori_loop(0, NCH, p2chunk, 0)

    pltpu.sync_copy(big, S_h.at[pl.ds(pl.multiple_of(n0, 16), NPT)])
    pltpu.sync_copy(denb, den_h.at[pl.ds(pl.multiple_of(n0, 16), NPT)])


def _edge_sc(ag, bg, am, bm, wg2, s_idx, n_idx, wp):
    f = pl.kernel(
        _edge_body,
        out_type=(
            jax.ShapeDtypeStruct((N2, H), jnp.float32),
            jax.ShapeDtypeStruct((N2,), jnp.float32),
            jax.ShapeDtypeStruct((NT * GCAP,), jnp.float32),
        ),
        mesh=_mesh(),
        compiler_params=_SC_PARAMS,
        scratch_types=[
            pltpu.VMEM((NPT, H), jnp.float32),
            pltpu.VMEM((CHUNK,), jnp.int32),
            pltpu.VMEM((CHUNK,), jnp.int32),
            pltpu.VMEM((CHUNK + L,), jnp.int32),
            pltpu.VMEM((CHUNK + L,), jnp.int32),
            pltpu.VMEM((2, L, H), jnp.float32),
            pltpu.VMEM((CHUNK + L,), jnp.float32),
            pltpu.VMEM((H,), jnp.float32),
            pltpu.VMEM((NPT,), jnp.float32),
            pltpu.VMEM((NPT,), jnp.float32),
            pltpu.VMEM((N2,), jnp.float32),
            pltpu.SemaphoreType.DMA,
            pltpu.SemaphoreType.DMA,
        ],
        interpret=INTERPRET,
    )
    S, den, _ = f(ag, bg, am, bm, wg2, s_idx, n_idx, wp)
    return S, den


def _cry_body(cidx_h, glog_h, wpc_h, hm_h, S_h, den_h,
              idxb, glb, wpb, hmb, Sloc, gmx, denb, sem):
    cix = lax.axis_index("c")
    six = lax.axis_index("s")
    wid = cix * 16 + six
    c0 = wid * CPT
    iota = lax.iota(jnp.int32, L)

    _fill1d(gmx, 48, NEG)
    _fill1d(denb, 48, 0.0)
    _zero2d(Sloc, CPT)

    def bchunk(c, carry):
        i0, i1 = carry
        pltpu.sync_copy(cidx_h.at[pl.ds(pl.multiple_of(c * CHUNK, 16), CHUNK)], idxb)

        def bbody(i, cc):
            a0, a1 = cc
            v = idxb[pl.ds(i * L, L)]
            one = jnp.full((L,), 1.0, jnp.float32)
            zero = jnp.full((L,), 0.0, jnp.float32)
            a0 = a0 + jnp.sum(jnp.where(v < c0, one, zero)).astype(jnp.int32)
            a1 = a1 + jnp.sum(jnp.where(v < c0 + CPT, one, zero)).astype(jnp.int32)
            return (a0, a1)

        return lax.fori_loop(0, CHUNK // L, bbody, (i0, i1))

    i0, i1 = lax.fori_loop(0, N_NODES // CHUNK, bchunk, (0, 0))
    g0 = i0 // L
    g1 = (i1 + L - 1) // L

    def p1(g, _):
        pos = g * L
        pltpu.sync_copy(cidx_h.at[pl.ds(pl.multiple_of(pos, 16), L)], idxb.at[pl.ds(0, L)])
        pltpu.sync_copy(glog_h.at[pl.ds(pl.multiple_of(pos, 16), L)], glb.at[pl.ds(0, L)])
        idx16 = idxb[pl.ds(0, L)]
        gl16 = glb[pl.ds(0, L)]
        for j in range(L):
            cj = idx16[j]

            @pl.when((cj >= c0) & (cj < c0 + CPT))
            def _():
                spl = jnp.full((L,), cj - c0, jnp.int32)
                old = plsc.load_gather(gmx, [spl])
                plsc.store_scatter(
                    gmx, [spl],
                    jnp.maximum(old, jnp.full((L,), gl16[j], jnp.float32)))
        return 0

    lax.fori_loop(g0, g1, p1, 0)

    def p2(g, _):
        pos = g * L
        pltpu.sync_copy(cidx_h.at[pl.ds(pl.multiple_of(pos, 16), L)], idxb.at[pl.ds(0, L)])
        pltpu.sync_copy(glog_h.at[pl.ds(pl.multiple_of(pos, 16), L)], glb.at[pl.ds(0, L)])
        pltpu.sync_copy(wpc_h.at[pl.ds(pl.multiple_of(pos, 16), L)], wpb.at[pl.ds(0, L)])
        pltpu.sync_copy(hm_h.at[pl.ds(pl.multiple_of(pos, 16), L)], hmb)
        idx16 = idxb[pl.ds(0, L)]
        gl16 = glb[pl.ds(0, L)]
        wp16 = wpb[pl.ds(0, L)]
        m = (idx16 >= c0) & (idx16 < c0 + CPT)
        idxl = jnp.where(m, idx16 - c0, jnp.zeros((L,), jnp.int32))
        gm16 = plsc.load_gather(gmx, [idxl])
        p16 = jnp.where(m, wp16 * jnp.exp(gl16 - gm16),
                        jnp.zeros((L,), jnp.float32))
        plsc.addupdate_scatter(denb, [idxl], p16)
        for j in range(L):
            cj = idxl[j]
            pj = jnp.full((L,), p16[j], jnp.float32)
            for k in range(H // L):
                plsc.addupdate(Sloc.at[cj, pl.ds(k * L, L)],
                               hmb[j, pl.ds(k * L, L)] * pj)
        return 0

    lax.fori_loop(g0, g1, p2, 0)

    pltpu.sync_copy(Sloc, S_h.at[pl.ds(pl.multiple_of(c0, 8), CPT)])
    pltpu.sync_copy(denb.at[pl.ds(0, CPT)], den_h.at[pl.ds(pl.multiple_of(c0, 8), CPT)])


def _cry_sc(cidx, glog, wpc, hm):
    f = pl.kernel(
        _cry_body,
        out_type=(
            jax.ShapeDtypeStruct((NCRY2, H), jnp.float32),
            jax.ShapeDtypeStruct((NCRY2,), jnp.float32),
        ),
        mesh=_mesh(),
        compiler_params=_SC_PARAMS,
        scratch_types=[
            pltpu.VMEM((CHUNK,), jnp.int32),
            pltpu.VMEM((CHUNK,), jnp.float32),
            pltpu.VMEM((CHUNK,), jnp.float32),
            pltpu.VMEM((L, H), jnp.float32),
            pltpu.VMEM((CPT, H), jnp.float32),
            pltpu.VMEM((48,), jnp.float32),
            pltpu.VMEM((48,), jnp.float32),
            pltpu.SemaphoreType.DMA,
        ],
        interpret=INTERPRET,
    )
    return f(cidx, glog, wpc, hm)


def _tc_call(body, out_shapes, n_in, grid, in_specs, out_specs):
    return pl.pallas_call(
        body,
        grid=grid,
        in_specs=in_specs,
        out_specs=out_specs,
        out_shape=out_shapes,
        interpret=INTERPRET,
    )


def _embed_body(ef_ref, sf_ref, ew_ref, we_ref, be_ref, wsm_ref, wsw_ref,
                bs_ref, pows_ref, fea_ref, wp_ref):
    ef = ef_ref[...]
    sf = sf_ref[...]
    ew = ew_ref[...]
    elem = ef @ we_ref[...] + be_ref[...]
    sym = sf @ wsm_ref[...] + ew * wsw_ref[...] + bs_ref[...]
    fea_ref[...] = jnp.concatenate([elem, sym], axis=1)
    wp_ref[...] = jnp.exp(jnp.log(ew) * pows_ref[...])


def _embed_tc(ef, sf, ew, we, be, wsm, wsw, bs, pows):
    blk = 1024
    g = N2 // blk
    return pl.pallas_call(
        _embed_body,
        grid=(g,),
        in_specs=[
            pl.BlockSpec((blk, 128), lambda i: (i, 0)),
            pl.BlockSpec((blk, 128), lambda i: (i, 0)),
            pl.BlockSpec((blk, 1), lambda i: (i, 0)),
            pl.BlockSpec((128, 32), lambda i: (0, 0)),
            pl.BlockSpec((1, 32), lambda i: (0, 0)),
            pl.BlockSpec((128, 32), lambda i: (0, 0)),
            pl.BlockSpec((1, 32), lambda i: (0, 0)),
            pl.BlockSpec((1, 32), lambda i: (0, 0)),
            pl.BlockSpec((1, 8), lambda i: (0, 0)),
        ],
        out_specs=[
            pl.BlockSpec((blk, F), lambda i: (i, 0)),
            pl.BlockSpec((blk, 8), lambda i: (i, 0)),
        ],
        out_shape=[
            jax.ShapeDtypeStruct((N2, F), jnp.float32),
            jax.ShapeDtypeStruct((N2, 8), jnp.float32),
        ],
        interpret=INTERPRET,
    )(ef, sf, ew, we, be, wsm, wsw, bs, pows)


def _tables_body(fea_ref, wga_ref, wgb_ref, bg_ref, wma_ref, wmb_ref, bm_ref,
                 ag_ref, bgt_ref, am_ref, bmt_ref):
    fea = fea_ref[...]
    ag_ref[...] = fea @ wga_ref[...] + bg_ref[...]
    bgt_ref[...] = fea @ wgb_ref[...]
    am_ref[...] = fea @ wma_ref[...] + bm_ref[...]
    bmt_ref[...] = fea @ wmb_ref[...]


def _tables_tc(fea, wga, wgb, bg, wma, wmb, bm):
    blk = 1024
    return pl.pallas_call(
        _tables_body,
        grid=(N2 // blk,),
        in_specs=[
            pl.BlockSpec((blk, F), lambda i: (i, 0)),
            pl.BlockSpec((F, H), lambda i: (0, 0)),
            pl.BlockSpec((F, H), lambda i: (0, 0)),
            pl.BlockSpec((1, H), lambda i: (0, 0)),
            pl.BlockSpec((F, H), lambda i: (0, 0)),
            pl.BlockSpec((F, H), lambda i: (0, 0)),
            pl.BlockSpec((1, H), lambda i: (0, 0)),
        ],
        out_specs=[pl.BlockSpec((blk, H), lambda i: (i, 0))] * 4,
        out_shape=[jax.ShapeDtypeStruct((N2, H), jnp.float32)] * 4,
        interpret=INTERPRET,
    )(fea, wga, wgb, bg, wma, wmb, bm)


def _update_body(S_ref, den_ref, fea_ref, w2_ref, b2_ref, out_ref):
    den = den_ref[...]
    fr = 1.0 / (den + 1e-10)
    out_ref[...] = ((S_ref[...] @ w2_ref[...]) * fr
                    + b2_ref[...] * (den * fr) + fea_ref[...])


def _update_tc(S, den, fea, w2, b2):
    blk = 1024
    return pl.pallas_call(
        _update_body,
        grid=(N2 // blk,),
        in_specs=[
            pl.BlockSpec((blk, H), lambda i: (i, 0)),
            pl.BlockSpec((blk, 1), lambda i: (i, 0)),
            pl.BlockSpec((blk, F), lambda i: (i, 0)),
            pl.BlockSpec((H, F), lambda i: (0, 0)),
            pl.BlockSpec((1, F), lambda i: (0, 0)),
        ],
        out_specs=pl.BlockSpec((blk, F), lambda i: (i, 0)),
        out_shape=jax.ShapeDtypeStruct((N2, F), jnp.float32),
        interpret=INTERPRET,
    )(S, den, fea, w2, b2)


def _cryfeat_body(fea_ref, wg1_ref, bg1_ref, wg2_ref, wm1_ref, bm1_ref,
                  gl_ref, hm_ref):
    fea = fea_ref[...]
    hg = fea @ wg1_ref[...] + bg1_ref[...]
    hg = jnp.maximum(hg, 0.01 * hg)
    gl_ref[...] = hg @ wg2_ref[...]
    hm = fea @ wm1_ref[...] + bm1_ref[...]
    hm_ref[...] = jnp.maximum(hm, 0.01 * hm)


def _cryfeat_tc(fea, wg1, bg1, wg2, wm1, bm1):
    blk = 1024
    return pl.pallas_call(
        _cryfeat_body,
        grid=(N2 // blk,),
        in_specs=[
            pl.BlockSpec((blk, F), lambda i: (i, 0)),
            pl.BlockSpec((F, H), lambda i: (0, 0)),
            pl.BlockSpec((1, H), lambda i: (0, 0)),
            pl.BlockSpec((H, 1), lambda i: (0, 0)),
            pl.BlockSpec((F, H), lambda i: (0, 0)),
            pl.BlockSpec((1, H), lambda i: (0, 0)),
        ],
        out_specs=[
            pl.BlockSpec((blk, 1), lambda i: (i, 0)),
            pl.BlockSpec((blk, H), lambda i: (i, 0)),
        ],
        out_shape=[
            jax.ShapeDtypeStruct((N2, 1), jnp.float32),
            jax.ShapeDtypeStruct((N2, H), jnp.float32),
        ],
        interpret=INTERPRET,
    )(fea, wg1, bg1, wg2, wm1, bm1)


def _aug_body(S_ref, den_ref, w2_ref, b2_ref, aug_ref, out_ref):
    den = den_ref[...]
    fr = 1.0 / (den + 1e-10)
    cry = (S_ref[...] @ w2_ref[...]) * fr + b2_ref[...] * (den * fr)
    aug = aug_ref[0:1, :]
    rows = lax.broadcasted_iota(jnp.int32, (N_AUG, NCRY2), 0)
    onehot = jnp.where(rows == aug, 1.0, 0.0)
    summed = onehot @ cry
    counts = jnp.sum(onehot, axis=1, keepdims=True)
    out_ref[...] = summed / jnp.maximum(counts, 1.0)


def _aug_tc(S, den, w2, b2, aug):
    return pl.pallas_call(
        _aug_body,
        grid=(1,),
        in_specs=[
            pl.BlockSpec((NCRY2, H), lambda i: (0, 0)),
            pl.BlockSpec((NCRY2, 1), lambda i: (0, 0)),
            pl.BlockSpec((H, F), lambda i: (0, 0)),
            pl.BlockSpec((1, F), lambda i: (0, 0)),
            pl.BlockSpec((8, NCRY2), lambda i: (0, 0)),
        ],
        out_specs=pl.BlockSpec((N_AUG, F), lambda i: (0, 0)),
        out_shape=jax.ShapeDtypeStruct((N_AUG, F), jnp.float32),
        interpret=INTERPRET,
    )(S, den, w2, b2, aug)


def kernel(elem_weights, elem_fea, sym_fea, self_idx, nbr_idx, cry_elem_idx,
           aug_cry_idx, params):
    pad_n = N2 - N_NODES
    ef = jnp.pad(elem_fea, ((0, pad_n), (0, 0)))
    sf = jnp.pad(sym_fea, ((0, pad_n), (0, 0)))
    ew = jnp.pad(elem_weights, ((0, pad_n), (0, 0)), constant_values=1.0)

    we, be = params["elem_embed"]
    ws, bs = params["sym_embed"]
    wsm = ws[:128]
    wsw = ws[128:129]

    g_pows = [lyr["heads"][0]["pow"] for lyr in params["graphs"]]
    c_pow = params["cry_pool"][0]["pow"]
    pows = jnp.concatenate(g_pows + [c_pow, jnp.zeros((4,), jnp.float32)])
    pows = pows.reshape(1, 8)

    fea, wp8 = _embed_tc(ef, sf, ew, we, be.reshape(1, 32), wsm, wsw,
                         bs.reshape(1, 32), pows)

    for li, lyr in enumerate(params["graphs"]):
        p = lyr["heads"][0]
        (wg1, bg1), = p["gate"]["hidden"]
        wg2, _ = p["gate"]["out"]
        (wm1, bm1), = p["message"]["hidden"]
        wm2, bm2 = p["message"]["out"]
        ag, bgt, am, bmt = _tables_tc(
            fea, wg1[:F], wg1[F:], bg1.reshape(1, H),
            wm1[:F], wm1[F:], bm1.reshape(1, H))
        wp_l = wp8[:, li] + 0.0
        S, den = _edge_sc(ag, bgt, am, bmt,
                          wg2[:, 0] + 0.0,
                          self_idx, nbr_idx, wp_l)
        fea = _update_tc(S, den.reshape(N2, 1), fea, wm2, bm2.reshape(1, F))

    p = params["cry_pool"][0]
    (wg1, bg1), = p["gate"]["hidden"]
    wg2, _ = p["gate"]["out"]
    (wm1, bm1), = p["message"]["hidden"]
    wm2, bm2 = p["message"]["out"]
    glog, hm = _cryfeat_tc(fea, wg1, bg1.reshape(1, H), wg2,
                           wm1, bm1.reshape(1, H))
    wpc = wp8[:, 3] + 0.0
    Scry, dencry = _cry_sc(cry_elem_idx, glog.reshape(N2), wpc, hm)

    aug_pad = jnp.full((NCRY2,), 1 << 20, jnp.int32).at[:N_CRY].set(aug_cry_idx)
    aug_pad = jnp.broadcast_to(aug_pad.reshape(1, NCRY2), (8, NCRY2))
    return _aug_tc(Scry, dencry.reshape(NCRY2, 1), wm2, bm2.reshape(1, F),
                   aug_pad)


# speedup vs baseline: 1.3796x; 1.2042x over previous
import functools
import jax
import jax.numpy as jnp
from jax import lax
from jax.experimental import pallas as pl
from jax.experimental.pallas import tpu as pltpu
from jax.experimental.pallas import tpu_sc as plsc

INTERPRET = False

N_NODES = 10000
N2 = 10240
NPT = 320
N_EDGES = 160000
CHUNK = 2000
NCH = N_EDGES // CHUNK
GCAP = 163840
N_CRY = 1250
NCRY2 = 1280
CPT = 40
N_AUG = 160
F = 64
H = 256
NT = 32
L = 16
NEG = -3.0e38


def _mesh():
    return plsc.VectorSubcoreMesh(core_axis_name="c", subcore_axis_name="s")


_SC_PARAMS = pltpu.CompilerParams(needs_layout_passes=False)


def _zero2d(ref, rows):
    def zrow(i, _):
        for k in range(H // L):
            ref[i, pl.ds(k * L, L)] = jnp.zeros((L,), jnp.float32)
        return 0
    lax.fori_loop(0, rows, zrow, 0)


def _fill1d(ref, n, val):
    def zb(i, _):
        ref[pl.ds(i * L, L)] = jnp.full((L,), val, jnp.float32)
        return 0
    lax.fori_loop(0, n // L, zb, 0)


BLOCK = 2048
NGB = BLOCK // L


def _part_body(s_h, n_h, wls_h, wln_h, cnt_h,
               s_in, n_in, wl_sb, wl_nb, cb, sem):
    cix = lax.axis_index("c")
    six = lax.axis_index("s")
    wid = cix * 16 + six
    n0 = wid * NPT
    base = wid * GCAP
    sent = jnp.where(n0 + NPT >= N2, 0, n0 + NPT)

    def chunk(c, gtot):
        pltpu.sync_copy(s_h.at[pl.ds(pl.multiple_of(c * CHUNK, 16), CHUNK)],
                        s_in)
        pltpu.sync_copy(n_h.at[pl.ds(pl.multiple_of(c * CHUNK, 16), CHUNK)],
                        n_in)

        def cbody(i, off):
            vs = s_in[pl.ds(i * L, L)]
            vn = n_in[pl.ds(i * L, L)]
            m = (vs >= n0) & (vs < n0 + NPT)
            plsc.store_compressed(wl_sb.at[pl.ds(off, L)], vs, mask=m)
            plsc.store_compressed(wl_nb.at[pl.ds(off, L)], vn, mask=m)
            cnt = jnp.sum(jnp.where(m, jnp.full((L,), 1.0, jnp.float32),
                                    jnp.full((L,), 0.0, jnp.float32)))
            return off + cnt.astype(jnp.int32)

        off = lax.fori_loop(0, CHUNK // L, cbody, 0)
        wl_sb[pl.ds(off, L)] = jnp.zeros((L,), jnp.int32) + sent
        wl_nb[pl.ds(off, L)] = jnp.zeros((L,), jnp.int32)
        offp = ((off + L - 1) // L) * L
        pltpu.sync_copy(
            wl_sb, wls_h.at[pl.ds(pl.multiple_of(base + gtot, 16), BLOCK)])
        pltpu.sync_copy(
            wl_nb, wln_h.at[pl.ds(pl.multiple_of(base + gtot, 16), BLOCK)])
        return gtot + offp

    gtot = lax.fori_loop(0, NCH, chunk, 0)
    cb[pl.ds(0, L)] = jnp.zeros((L,), jnp.int32) + gtot
    pltpu.sync_copy(cb, cnt_h.at[pl.ds(pl.multiple_of(wid * L, 16), L)])


def _part_sc(s_idx, n_idx):
    f = pl.kernel(
        _part_body,
        out_type=(
            jax.ShapeDtypeStruct((NT * GCAP,), jnp.int32),
            jax.ShapeDtypeStruct((NT * GCAP,), jnp.int32),
            jax.ShapeDtypeStruct((NT * L,), jnp.int32),
        ),
        mesh=_mesh(),
        compiler_params=_SC_PARAMS,
        scratch_types=[
            pltpu.VMEM((CHUNK,), jnp.int32),
            pltpu.VMEM((CHUNK,), jnp.int32),
            pltpu.VMEM((BLOCK,), jnp.int32),
            pltpu.VMEM((BLOCK,), jnp.int32),
            pltpu.VMEM((L,), jnp.int32),
            pltpu.SemaphoreType.DMA,
        ],
        interpret=INTERPRET,
    )
    return f(s_idx, n_idx)


def _edge_body(ag_h, bg_h, am_h, bm_h, wg2_h, wls_h, wln_h, cnt_h, wp_h,
               S_h, den_h, glog_h,
               big, wl_sb, wl_nb, gbufA, gbufB, glogb, wg2b, gmaxb, denb,
               wpv, cb, sA0, sA1, sB0, sB1):
    cix = lax.axis_index("c")
    six = lax.axis_index("s")
    wid = cix * 16 + six
    n0 = wid * NPT
    base = wid * GCAP
    iota = lax.iota(jnp.int32, L)

    pltpu.sync_copy(wg2_h, wg2b)
    pltpu.sync_copy(wp_h, wpv)
    pltpu.sync_copy(ag_h.at[pl.ds(pl.multiple_of(n0, 16), NPT)], big)
    pltpu.sync_copy(cnt_h.at[pl.ds(pl.multiple_of(wid * L, 16), L)], cb)
    _fill1d(gmaxb, NPT, NEG)
    _fill1d(denb, NPT, 0.0)
    ntot = cb[pl.ds(0, L)][0]
    ngrp = ntot // L
    nblk = (ngrp + NGB - 1) // NGB

    def stage_wl(b):
        pltpu.sync_copy(
            wls_h.at[pl.ds(pl.multiple_of(base + b * BLOCK, 16), BLOCK)],
            wl_sb)
        pltpu.sync_copy(
            wln_h.at[pl.ds(pl.multiple_of(base + b * BLOCK, 16), BLOCK)],
            wl_nb)

    def sv_of(g):
        vs = wl_sb[pl.ds(g * L, L)]
        valid = (vs >= n0) & (vs < n0 + NPT)
        sv = jnp.clip(vs - n0, 0, NPT - 1)
        return vs, valid, sv

    def issue1(g, sem_, buf):
        pltpu.async_copy(bg_h.at[wl_nb.at[pl.ds(g * L, L)]],
                         gbufA.at[buf], sem_)

    def wait1(sem_, buf):
        pltpu.make_async_copy(bg_h.at[wl_nb.at[pl.ds(0, L)]],
                              gbufA.at[buf], sem_).wait()

    def proc1(g, buf):
        vs, valid, sv = sv_of(g)
        vglog = jnp.zeros((L,), jnp.float32)
        for j in range(L):
            sj = sv[j]
            dacc = jnp.zeros((L,), jnp.float32)
            for k in range(H // L):
                va = big[sj, pl.ds(k * L, L)]
                vb = gbufA[buf, j, pl.ds(k * L, L)]
                hh = va + vb
                r = jnp.maximum(hh, 0.01 * hh)
                dacc = dacc + r * wg2b[pl.ds(k * L, L)]
            gl = jnp.sum(dacc)

            @pl.when(vs[j] < n0 + NPT)
            def _():
                spl = jnp.full((L,), sj, jnp.int32)
                old = plsc.load_gather(gmaxb, [spl])
                plsc.store_scatter(
                    gmaxb, [spl],
                    jnp.maximum(old, jnp.full((L,), gl, jnp.float32)))

            vglog = jnp.where(iota == j, gl, vglog)
        glogb[pl.ds(g * L, L)] = vglog

    def p1blk(b, _):
        stage_wl(b)
        ngb = jnp.minimum(ngrp - b * NGB, NGB)
        issue1(0, sA0, 0)

        def tbody(t, _):
            g0 = 2 * t
            g1 = jnp.minimum(2 * t + 1, ngb - 1)
            g2 = jnp.minimum(2 * t + 2, ngb - 1)
            issue1(g1, sA1, 1)
            wait1(sA0, 0)
            proc1(g0, 0)
            issue1(g2, sA0, 0)
            wait1(sA1, 1)

            @pl.when(g1 > g0)
            def _():
                proc1(g1, 1)

            return 0

        lax.fori_loop(0, (ngb + 1) // 2, tbody, 0)
        wait1(sA0, 0)
        pltpu.sync_copy(
            glogb, glog_h.at[pl.ds(pl.multiple_of(base + b * BLOCK, 16),
                                   BLOCK)])
        return 0

    lax.fori_loop(0, nblk, p1blk, 0)

    _zero2d(big, NPT)

    def issue2(g, semA, semB, buf):
        pltpu.async_copy(am_h.at[wl_sb.at[pl.ds(g * L, L)]],
                         gbufA.at[buf], semA)
        pltpu.async_copy(bm_h.at[wl_nb.at[pl.ds(g * L, L)]],
                         gbufB.at[buf], semB)

    def wait2(semA, semB, buf):
        pltpu.make_async_copy(am_h.at[wl_sb.at[pl.ds(0, L)]],
                              gbufA.at[buf], semA).wait()
        pltpu.make_async_copy(bm_h.at[wl_nb.at[pl.ds(0, L)]],
                              gbufB.at[buf], semB).wait()

    def proc2(g, buf):
        vs, valid, sv = sv_of(g)
        nv = wl_nb[pl.ds(g * L, L)]
        gl16 = glogb[pl.ds(g * L, L)]
        gm16 = plsc.load_gather(gmaxb, [sv])
        wp16 = plsc.load_gather(wpv, [nv])
        p16 = jnp.where(valid, wp16 * jnp.exp(gl16 - gm16),
                        jnp.zeros((L,), jnp.float32))
        plsc.addupdate_scatter(denb, [sv], p16)
        for j in range(L):
            sj = sv[j]
            pj = jnp.full((L,), p16[j], jnp.float32)
            for k in range(H // L):
                am = gbufA[buf, j, pl.ds(k * L, L)]
                bm = gbufB[buf, j, pl.ds(k * L, L)]
                hh = am + bm
                r = jnp.maximum(hh, 0.01 * hh)
                plsc.addupdate(big.at[sj, pl.ds(k * L, L)], r * pj)

    def p2blk(b, _):
        stage_wl(b)
        pltpu.sync_copy(
            glog_h.at[pl.ds(pl.multiple_of(base + b * BLOCK, 16), BLOCK)],
            glogb)
        ngb = jnp.minimum(ngrp - b * NGB, NGB)
        issue2(0, sA0, sB0, 0)

        def tbody(t, _):
            g0 = 2 * t
            g1 = jnp.minimum(2 * t + 1, ngb - 1)
            g2 = jnp.minimum(2 * t + 2, ngb - 1)
            issue2(g1, sA1, sB1, 1)
            wait2(sA0, sB0, 0)
            proc2(g0, 0)
            issue2(g2, sA0, sB0, 0)
            wait2(sA1, sB1, 1)

            @pl.when(g1 > g0)
            def _():
                proc2(g1, 1)

            return 0

        lax.fori_loop(0, (ngb + 1) // 2, tbody, 0)
        wait2(sA0, sB0, 0)
        return 0

    lax.fori_loop(0, nblk, p2blk, 0)

    pltpu.sync_copy(big, S_h.at[pl.ds(pl.multiple_of(n0, 16), NPT)])
    pltpu.sync_copy(denb, den_h.at[pl.ds(pl.multiple_of(n0, 16), NPT)])


def _edge_sc(ag, bg, am, bm, wg2, wls, wln, cnt, wp):
    f = pl.kernel(
        _edge_body,
        out_type=(
            jax.ShapeDtypeStruct((N2, H), jnp.float32),
            jax.ShapeDtypeStruct((N2,), jnp.float32),
            jax.ShapeDtypeStruct((NT * GCAP,), jnp.float32),
        ),
        mesh=_mesh(),
        compiler_params=_SC_PARAMS,
        scratch_types=[
            pltpu.VMEM((NPT, H), jnp.float32),
            pltpu.VMEM((BLOCK,), jnp.int32),
            pltpu.VMEM((BLOCK,), jnp.int32),
            pltpu.VMEM((2, L, H), jnp.float32),
            pltpu.VMEM((2, L, H), jnp.float32),
            pltpu.VMEM((BLOCK,), jnp.float32),
            pltpu.VMEM((H,), jnp.float32),
            pltpu.VMEM((NPT,), jnp.float32),
            pltpu.VMEM((NPT,), jnp.float32),
            pltpu.VMEM((N2,), jnp.float32),
            pltpu.VMEM((L,), jnp.int32),
            pltpu.SemaphoreType.DMA,
            pltpu.SemaphoreType.DMA,
            pltpu.SemaphoreType.DMA,
            pltpu.SemaphoreType.DMA,
        ],
        interpret=INTERPRET,
    )
    S, den, _ = f(ag, bg, am, bm, wg2, wls, wln, cnt, wp)
    return S, den


def _cry_body(cidx_h, glog_h, wpc_h, hm_h, S_h, den_h,
              idxb, glb, wpb, hmb, Sloc, gmx, denb, sem):
    cix = lax.axis_index("c")
    six = lax.axis_index("s")
    wid = cix * 16 + six
    c0 = wid * CPT
    iota = lax.iota(jnp.int32, L)

    _fill1d(gmx, 48, NEG)
    _fill1d(denb, 48, 0.0)
    _zero2d(Sloc, CPT)

    def bchunk(c, carry):
        i0, i1 = carry
        pltpu.sync_copy(cidx_h.at[pl.ds(pl.multiple_of(c * CHUNK, 16), CHUNK)], idxb)

        def bbody(i, cc):
            a0, a1 = cc
            v = idxb[pl.ds(i * L, L)]
            one = jnp.full((L,), 1.0, jnp.float32)
            zero = jnp.full((L,), 0.0, jnp.float32)
            a0 = a0 + jnp.sum(jnp.where(v < c0, one, zero)).astype(jnp.int32)
            a1 = a1 + jnp.sum(jnp.where(v < c0 + CPT, one, zero)).astype(jnp.int32)
            return (a0, a1)

        return lax.fori_loop(0, CHUNK // L, bbody, (i0, i1))

    i0, i1 = lax.fori_loop(0, N_NODES // CHUNK, bchunk, (0, 0))
    g0 = i0 // L
    g1 = (i1 + L - 1) // L

    def p1(g, _):
        pos = g * L
        pltpu.sync_copy(cidx_h.at[pl.ds(pl.multiple_of(pos, 16), L)], idxb.at[pl.ds(0, L)])
        pltpu.sync_copy(glog_h.at[pl.ds(pl.multiple_of(pos, 16), L)], glb.at[pl.ds(0, L)])
        idx16 = idxb[pl.ds(0, L)]
        gl16 = glb[pl.ds(0, L)]
        for j in range(L):
            cj = idx16[j]

            @pl.when((cj >= c0) & (cj < c0 + CPT))
            def _():
                spl = jnp.full((L,), cj - c0, jnp.int32)
                old = plsc.load_gather(gmx, [spl])
                plsc.store_scatter(
                    gmx, [spl],
                    jnp.maximum(old, jnp.full((L,), gl16[j], jnp.float32)))
        return 0

    lax.fori_loop(g0, g1, p1, 0)

    def p2(g, _):
        pos = g * L
        pltpu.sync_copy(cidx_h.at[pl.ds(pl.multiple_of(pos, 16), L)], idxb.at[pl.ds(0, L)])
        pltpu.sync_copy(glog_h.at[pl.ds(pl.multiple_of(pos, 16), L)], glb.at[pl.ds(0, L)])
        pltpu.sync_copy(wpc_h.at[pl.ds(pl.multiple_of(pos, 16), L)], wpb.at[pl.ds(0, L)])
        pltpu.sync_copy(hm_h.at[pl.ds(pl.multiple_of(pos, 16), L)], hmb)
        idx16 = idxb[pl.ds(0, L)]
        gl16 = glb[pl.ds(0, L)]
        wp16 = wpb[pl.ds(0, L)]
        m = (idx16 >= c0) & (idx16 < c0 + CPT)
        idxl = jnp.where(m, idx16 - c0, jnp.zeros((L,), jnp.int32))
        gm16 = plsc.load_gather(gmx, [idxl])
        p16 = jnp.where(m, wp16 * jnp.exp(gl16 - gm16),
                        jnp.zeros((L,), jnp.float32))
        plsc.addupdate_scatter(denb, [idxl], p16)
        for j in range(L):
            cj = idxl[j]
            pj = jnp.full((L,), p16[j], jnp.float32)
            for k in range(H // L):
                plsc.addupdate(Sloc.at[cj, pl.ds(k * L, L)],
                               hmb[j, pl.ds(k * L, L)] * pj)
        return 0

    lax.fori_loop(g0, g1, p2, 0)

    pltpu.sync_copy(Sloc, S_h.at[pl.ds(pl.multiple_of(c0, 8), CPT)])
    pltpu.sync_copy(denb.at[pl.ds(0, CPT)], den_h.at[pl.ds(pl.multiple_of(c0, 8), CPT)])


def _cry_sc(cidx, glog, wpc, hm):
    f = pl.kernel(
        _cry_body,
        out_type=(
            jax.ShapeDtypeStruct((NCRY2, H), jnp.float32),
            jax.ShapeDtypeStruct((NCRY2,), jnp.float32),
        ),
        mesh=_mesh(),
        compiler_params=_SC_PARAMS,
        scratch_types=[
            pltpu.VMEM((CHUNK,), jnp.int32),
            pltpu.VMEM((CHUNK,), jnp.float32),
            pltpu.VMEM((CHUNK,), jnp.float32),
            pltpu.VMEM((L, H), jnp.float32),
            pltpu.VMEM((CPT, H), jnp.float32),
            pltpu.VMEM((48,), jnp.float32),
            pltpu.VMEM((48,), jnp.float32),
            pltpu.SemaphoreType.DMA,
        ],
        interpret=INTERPRET,
    )
    return f(cidx, glog, wpc, hm)


def _tc_call(body, out_shapes, n_in, grid, in_specs, out_specs):
    return pl.pallas_call(
        body,
        grid=grid,
        in_specs=in_specs,
        out_specs=out_specs,
        out_shape=out_shapes,
        interpret=INTERPRET,
    )


def _embed_body(ef_ref, sf_ref, ew_ref, we_ref, be_ref, wsm_ref, wsw_ref,
                bs_ref, pows_ref, fea_ref, wp_ref):
    ef = ef_ref[...]
    sf = sf_ref[...]
    ew = ew_ref[...]
    elem = ef @ we_ref[...] + be_ref[...]
    sym = sf @ wsm_ref[...] + ew * wsw_ref[...] + bs_ref[...]
    fea_ref[...] = jnp.concatenate([elem, sym], axis=1)
    wp_ref[...] = jnp.exp(jnp.log(ew) * pows_ref[...])


def _embed_tc(ef, sf, ew, we, be, wsm, wsw, bs, pows):
    blk = 1024
    g = N2 // blk
    return pl.pallas_call(
        _embed_body,
        grid=(g,),
        in_specs=[
            pl.BlockSpec((blk, 128), lambda i: (i, 0)),
            pl.BlockSpec((blk, 128), lambda i: (i, 0)),
            pl.BlockSpec((blk, 1), lambda i: (i, 0)),
            pl.BlockSpec((128, 32), lambda i: (0, 0)),
            pl.BlockSpec((1, 32), lambda i: (0, 0)),
            pl.BlockSpec((128, 32), lambda i: (0, 0)),
            pl.BlockSpec((1, 32), lambda i: (0, 0)),
            pl.BlockSpec((1, 32), lambda i: (0, 0)),
            pl.BlockSpec((1, 8), lambda i: (0, 0)),
        ],
        out_specs=[
            pl.BlockSpec((blk, F), lambda i: (i, 0)),
            pl.BlockSpec((blk, 8), lambda i: (i, 0)),
        ],
        out_shape=[
            jax.ShapeDtypeStruct((N2, F), jnp.float32),
            jax.ShapeDtypeStruct((N2, 8), jnp.float32),
        ],
        interpret=INTERPRET,
    )(ef, sf, ew, we, be, wsm, wsw, bs, pows)


def _tables_body(fea_ref, wga_ref, wgb_ref, bg_ref, wma_ref, wmb_ref, bm_ref,
                 ag_ref, bgt_ref, am_ref, bmt_ref):
    fea = fea_ref[...]
    ag_ref[...] = fea @ wga_ref[...] + bg_ref[...]
    bgt_ref[...] = fea @ wgb_ref[...]
    am_ref[...] = fea @ wma_ref[...] + bm_ref[...]
    bmt_ref[...] = fea @ wmb_ref[...]


def _tables_tc(fea, wga, wgb, bg, wma, wmb, bm):
    blk = 1024
    return pl.pallas_call(
        _tables_body,
        grid=(N2 // blk,),
        in_specs=[
            pl.BlockSpec((blk, F), lambda i: (i, 0)),
            pl.BlockSpec((F, H), lambda i: (0, 0)),
            pl.BlockSpec((F, H), lambda i: (0, 0)),
            pl.BlockSpec((1, H), lambda i: (0, 0)),
            pl.BlockSpec((F, H), lambda i: (0, 0)),
            pl.BlockSpec((F, H), lambda i: (0, 0)),
            pl.BlockSpec((1, H), lambda i: (0, 0)),
        ],
        out_specs=[pl.BlockSpec((blk, H), lambda i: (i, 0))] * 4,
        out_shape=[jax.ShapeDtypeStruct((N2, H), jnp.float32)] * 4,
        interpret=INTERPRET,
    )(fea, wga, wgb, bg, wma, wmb, bm)


def _update_body(S_ref, den_ref, fea_ref, w2_ref, b2_ref, out_ref):
    den = den_ref[...]
    fr = 1.0 / (den + 1e-10)
    out_ref[...] = ((S_ref[...] @ w2_ref[...]) * fr
                    + b2_ref[...] * (den * fr) + fea_ref[...])


def _update_tc(S, den, fea, w2, b2):
    blk = 1024
    return pl.pallas_call(
        _update_body,
        grid=(N2 // blk,),
        in_specs=[
            pl.BlockSpec((blk, H), lambda i: (i, 0)),
            pl.BlockSpec((blk, 1), lambda i: (i, 0)),
            pl.BlockSpec((blk, F), lambda i: (i, 0)),
            pl.BlockSpec((H, F), lambda i: (0, 0)),
            pl.BlockSpec((1, F), lambda i: (0, 0)),
        ],
        out_specs=pl.BlockSpec((blk, F), lambda i: (i, 0)),
        out_shape=jax.ShapeDtypeStruct((N2, F), jnp.float32),
        interpret=INTERPRET,
    )(S, den, fea, w2, b2)


def _cryfeat_body(fea_ref, wg1_ref, bg1_ref, wg2_ref, wm1_ref, bm1_ref,
                  gl_ref, hm_ref):
    fea = fea_ref[...]
    hg = fea @ wg1_ref[...] + bg1_ref[...]
    hg = jnp.maximum(hg, 0.01 * hg)
    gl_ref[...] = hg @ wg2_ref[...]
    hm = fea @ wm1_ref[...] + bm1_ref[...]
    hm_ref[...] = jnp.maximum(hm, 0.01 * hm)


def _cryfeat_tc(fea, wg1, bg1, wg2, wm1, bm1):
    blk = 1024
    return pl.pallas_call(
        _cryfeat_body,
        grid=(N2 // blk,),
        in_specs=[
            pl.BlockSpec((blk, F), lambda i: (i, 0)),
            pl.BlockSpec((F, H), lambda i: (0, 0)),
            pl.BlockSpec((1, H), lambda i: (0, 0)),
            pl.BlockSpec((H, 1), lambda i: (0, 0)),
            pl.BlockSpec((F, H), lambda i: (0, 0)),
            pl.BlockSpec((1, H), lambda i: (0, 0)),
        ],
        out_specs=[
            pl.BlockSpec((blk, 1), lambda i: (i, 0)),
            pl.BlockSpec((blk, H), lambda i: (i, 0)),
        ],
        out_shape=[
            jax.ShapeDtypeStruct((N2, 1), jnp.float32),
            jax.ShapeDtypeStruct((N2, H), jnp.float32),
        ],
        interpret=INTERPRET,
    )(fea, wg1, bg1, wg2, wm1, bm1)


def _aug_body(S_ref, den_ref, w2_ref, b2_ref, aug_ref, out_ref):
    den = den_ref[...]
    fr = 1.0 / (den + 1e-10)
    cry = (S_ref[...] @ w2_ref[...]) * fr + b2_ref[...] * (den * fr)
    aug = aug_ref[0:1, :]
    rows = lax.broadcasted_iota(jnp.int32, (N_AUG, NCRY2), 0)
    onehot = jnp.where(rows == aug, 1.0, 0.0)
    summed = onehot @ cry
    counts = jnp.sum(onehot, axis=1, keepdims=True)
    out_ref[...] = summed / jnp.maximum(counts, 1.0)


def _aug_tc(S, den, w2, b2, aug):
    return pl.pallas_call(
        _aug_body,
        grid=(1,),
        in_specs=[
            pl.BlockSpec((NCRY2, H), lambda i: (0, 0)),
            pl.BlockSpec((NCRY2, 1), lambda i: (0, 0)),
            pl.BlockSpec((H, F), lambda i: (0, 0)),
            pl.BlockSpec((1, F), lambda i: (0, 0)),
            pl.BlockSpec((8, NCRY2), lambda i: (0, 0)),
        ],
        out_specs=pl.BlockSpec((N_AUG, F), lambda i: (0, 0)),
        out_shape=jax.ShapeDtypeStruct((N_AUG, F), jnp.float32),
        interpret=INTERPRET,
    )(S, den, w2, b2, aug)


def kernel(elem_weights, elem_fea, sym_fea, self_idx, nbr_idx, cry_elem_idx,
           aug_cry_idx, params):
    pad_n = N2 - N_NODES
    ef = jnp.pad(elem_fea, ((0, pad_n), (0, 0)))
    sf = jnp.pad(sym_fea, ((0, pad_n), (0, 0)))
    ew = jnp.pad(elem_weights, ((0, pad_n), (0, 0)), constant_values=1.0)

    we, be = params["elem_embed"]
    ws, bs = params["sym_embed"]
    wsm = ws[:128]
    wsw = ws[128:129]

    g_pows = [lyr["heads"][0]["pow"] for lyr in params["graphs"]]
    c_pow = params["cry_pool"][0]["pow"]
    pows = jnp.concatenate(g_pows + [c_pow, jnp.zeros((4,), jnp.float32)])
    pows = pows.reshape(1, 8)

    fea, wp8 = _embed_tc(ef, sf, ew, we, be.reshape(1, 32), wsm, wsw,
                         bs.reshape(1, 32), pows)
    wls, wln, cnt = _part_sc(self_idx, nbr_idx)

    for li, lyr in enumerate(params["graphs"]):
        p = lyr["heads"][0]
        (wg1, bg1), = p["gate"]["hidden"]
        wg2, _ = p["gate"]["out"]
        (wm1, bm1), = p["message"]["hidden"]
        wm2, bm2 = p["message"]["out"]
        ag, bgt, am, bmt = _tables_tc(
            fea, wg1[:F], wg1[F:], bg1.reshape(1, H),
            wm1[:F], wm1[F:], bm1.reshape(1, H))
        wp_l = wp8[:, li] + 0.0
        S, den = _edge_sc(ag, bgt, am, bmt,
                          wg2[:, 0] + 0.0,
                          wls, wln, cnt, wp_l)
        fea = _update_tc(S, den.reshape(N2, 1), fea, wm2, bm2.reshape(1, F))

    p = params["cry_pool"][0]
    (wg1, bg1), = p["gate"]["hidden"]
    wg2, _ = p["gate"]["out"]
    (wm1, bm1), = p["message"]["hidden"]
    wm2, bm2 = p["message"]["out"]
    glog, hm = _cryfeat_tc(fea, wg1, bg1.reshape(1, H), wg2,
                           wm1, bm1.reshape(1, H))
    wpc = wp8[:, 3] + 0.0
    Scry, dencry = _cry_sc(cry_elem_idx, glog.reshape(N2), wpc, hm)

    aug_pad = jnp.full((NCRY2,), 1 << 20, jnp.int32).at[:N_CRY].set(aug_cry_idx)
    aug_pad = jnp.broadcast_to(aug_pad.reshape(1, NCRY2), (8, NCRY2))
    return _aug_tc(Scry, dencry.reshape(NCRY2, 1), wm2, bm2.reshape(1, F),
                   aug_pad)
